# baseline (speedup 1.0000x reference)
import functools

import jax
import jax.numpy as jnp
from jax import lax
from jax.experimental import pallas as pl
from jax.experimental.pallas import tpu as pltpu
from jax.experimental.pallas import tpu_sc as plsc

VOCAB = 100000
TOKEN_DIM = 128
FEAT_DIM = 512
FEAT_EMB_DIM = 128
OUT_DIM = TOKEN_DIM + FEAT_EMB_DIM
B, L = 1024, 50
LP = 56
N_PAD_ROWS = B * LP

NC = 2
NS = 16
NW = NC * NS
BATCH_PER_W = B // NW
ROWS_PER_W = BATCH_PER_W * LP
CHUNK = 128
N_CHUNKS = ROWS_PER_W // CHUNK
NBUF = 4


def _sc_gather(table, idx_pad):
    mesh = plsc.VectorSubcoreMesh(core_axis_name="c", subcore_axis_name="s")

    @functools.partial(
        pl.kernel,
        mesh=mesh,
        out_type=jax.ShapeDtypeStruct((N_PAD_ROWS, TOKEN_DIM), jnp.float32),
        scratch_types=[
            pltpu.VMEM((ROWS_PER_W,), jnp.int32),
            pltpu.VMEM((NBUF, CHUNK, TOKEN_DIM), jnp.float32),
            [pltpu.SemaphoreType.DMA] * NBUF,
            [pltpu.SemaphoreType.DMA] * NBUF,
        ],
    )
    def gather_k(table_hbm, idx_hbm, out_hbm, idx_v, rows, gsems, wsems):
        wid = lax.axis_index("s") * NC + lax.axis_index("c")
        base = wid * ROWS_PER_W
        pltpu.sync_copy(idx_hbm.at[pl.ds(base, ROWS_PER_W)], idx_v)

        gathers = []
        writes = []
        for c in range(N_CHUNKS):
            off = c * CHUNK
            s = c % NBUF
            if c >= NBUF:
                writes[c - NBUF].wait()
            gcp = pltpu.make_async_copy(
                table_hbm.at[idx_v.at[pl.ds(off, CHUNK)]],
                rows.at[s],
                gsems[s],
            )
            gcp.start()
            gathers.append(gcp)
            if c >= 1:
                ps = (c - 1) % NBUF
                gathers[c - 1].wait()
                wcp = pltpu.make_async_copy(
                    rows.at[ps],
                    out_hbm.at[pl.ds(base + (c - 1) * CHUNK, CHUNK)],
                    wsems[ps],
                )
                wcp.start()
                writes.append(wcp)
        gathers[-1].wait()
        lcp = pltpu.make_async_copy(
            rows.at[(N_CHUNKS - 1) % NBUF],
            out_hbm.at[pl.ds(base + (N_CHUNKS - 1) * CHUNK, CHUNK)],
            wsems[(N_CHUNKS - 1) % NBUF],
        )
        lcp.start()
        writes.append(lcp)
        for c in range(max(0, N_CHUNKS - NBUF), N_CHUNKS):
            writes[c].wait()

    return gather_k(table, idx_pad)


BB = 64


def _mm_body(f_ref, w_ref, b_ref, o_ref):
    acc = lax.dot_general(
        f_ref[...],
        w_ref[...],
        (((2,), (1,)), ((), ())),
        preferred_element_type=jnp.float32,
    )
    o_ref[...] = acc + b_ref[...]


def _tc_project(features, W, b3d):
    return pl.pallas_call(
        _mm_body,
        grid=(B // BB,),
        in_specs=[
            pl.BlockSpec((BB, L, FEAT_DIM), lambda i: (i, 0, 0)),
            pl.BlockSpec((FEAT_EMB_DIM, FEAT_DIM), lambda i: (0, 0)),
            pl.BlockSpec((1, 1, FEAT_EMB_DIM), lambda i: (0, 0, 0)),
        ],
        out_specs=pl.BlockSpec((BB, L, FEAT_EMB_DIM), lambda i: (i, 0, 1)),
        out_shape=jax.ShapeDtypeStruct((B, L, OUT_DIM), jnp.float32),
    )(features, W, b3d)


def _ins_body(g_ref, _, o_ref):
    o_ref[...] = g_ref[:, :L, :]


def _tc_insert_tokens(gathered3d, partial_out):
    return pl.pallas_call(
        _ins_body,
        grid=(B // BB,),
        in_specs=[
            pl.BlockSpec((BB, LP, TOKEN_DIM), lambda i: (i, 0, 0)),
            pl.BlockSpec(memory_space=pl.ANY),
        ],
        out_specs=pl.BlockSpec((BB, L, TOKEN_DIM), lambda i: (i, 0, 0)),
        out_shape=jax.ShapeDtypeStruct((B, L, OUT_DIM), jnp.float32),
        input_output_aliases={1: 0},
    )(gathered3d, partial_out)


@jax.jit
def kernel(tokens, features, table, W, b):
    pad_idx = (
        jnp.arange(B, dtype=jnp.int32)[:, None] * (LP - L)
        + jnp.arange(LP - L, dtype=jnp.int32)[None, :]
    )
    idx_pad = jnp.concatenate([tokens.astype(jnp.int32), pad_idx], axis=1)
    gathered = _sc_gather(table, idx_pad.reshape(N_PAD_ROWS))
    partial = _tc_project(features, W, b.reshape(1, 1, FEAT_EMB_DIM))
    out = _tc_insert_tokens(gathered.reshape(B, LP, TOKEN_DIM), partial)
    return out


# speedup vs baseline: 1.9062x; 1.9062x over previous
import functools

import jax
import jax.numpy as jnp
from jax import lax
from jax.experimental import pallas as pl
from jax.experimental.pallas import tpu as pltpu
from jax.experimental.pallas import tpu_sc as plsc

VOCAB = 100000
TOKEN_DIM = 128
FEAT_DIM = 512
FEAT_EMB_DIM = 128
OUT_DIM = TOKEN_DIM + FEAT_EMB_DIM
B, L = 1024, 50
N_ROWS = B * L

NC = 2
NS = 16
NW = NC * NS
ROWS_PER_W = N_ROWS // NW
CHUNK = 128
NBUF = 4
SIZES = [CHUNK] * (ROWS_PER_W // CHUNK) + (
    [ROWS_PER_W % CHUNK] if ROWS_PER_W % CHUNK else []
)


def _sc_gather(table, idx_flat):
    mesh = plsc.VectorSubcoreMesh(core_axis_name="c", subcore_axis_name="s")

    @functools.partial(
        pl.kernel,
        mesh=mesh,
        out_type=jax.ShapeDtypeStruct((N_ROWS, TOKEN_DIM), jnp.float32),
        scratch_types=[
            pltpu.VMEM((ROWS_PER_W,), jnp.int32),
            pltpu.VMEM((NBUF, CHUNK, TOKEN_DIM), jnp.float32),
            [pltpu.SemaphoreType.DMA] * NBUF,
            [pltpu.SemaphoreType.DMA] * NBUF,
        ],
    )
    def gather_k(table_hbm, idx_hbm, out_hbm, idx_v, rows, gsems, wsems):
        wid = lax.axis_index("s") * NC + lax.axis_index("c")
        base = wid * ROWS_PER_W
        pltpu.sync_copy(idx_hbm.at[pl.ds(base, ROWS_PER_W)], idx_v)

        gathers = []
        writes = []
        for c, sz in enumerate(SIZES):
            off = c * CHUNK
            s = c % NBUF
            if c >= NBUF:
                writes[c - NBUF].wait()
            gcp = pltpu.make_async_copy(
                table_hbm.at[idx_v.at[pl.ds(off, sz)]],
                rows.at[s, pl.ds(0, sz)],
                gsems[s],
            )
            gcp.start()
            gathers.append(gcp)
            if c >= 1:
                ps = (c - 1) % NBUF
                psz = SIZES[c - 1]
                gathers[c - 1].wait()
                wcp = pltpu.make_async_copy(
                    rows.at[ps, pl.ds(0, psz)],
                    out_hbm.at[pl.ds(base + (c - 1) * CHUNK, psz)],
                    wsems[ps],
                )
                wcp.start()
                writes.append(wcp)
        n = len(SIZES)
        gathers[-1].wait()
        lcp = pltpu.make_async_copy(
            rows.at[(n - 1) % NBUF, pl.ds(0, SIZES[-1])],
            out_hbm.at[pl.ds(base + (n - 1) * CHUNK, SIZES[-1])],
            wsems[(n - 1) % NBUF],
        )
        lcp.start()
        writes.append(lcp)
        for c in range(max(0, n - NBUF), n):
            writes[c].wait()

    return gather_k(table, idx_flat)


def _mm_body(f_ref, w_ref, b_ref, o_ref):
    acc = lax.dot_general(
        f_ref[0],
        w_ref[...],
        (((1,), (1,)), ((), ())),
        preferred_element_type=jnp.float32,
    )
    o_ref[0] = acc + b_ref[...]


def _tc_project(featT, W, b2d):
    return pl.pallas_call(
        _mm_body,
        grid=(L,),
        in_specs=[
            pl.BlockSpec((1, B, FEAT_DIM), lambda i: (i, 0, 0)),
            pl.BlockSpec((FEAT_EMB_DIM, FEAT_DIM), lambda i: (0, 0)),
            pl.BlockSpec((1, FEAT_EMB_DIM), lambda i: (0, 0)),
        ],
        out_specs=pl.BlockSpec((1, B, FEAT_EMB_DIM), lambda i: (i, 0, 1)),
        out_shape=jax.ShapeDtypeStruct((L, B, OUT_DIM), jnp.float32),
    )(featT, W, b2d)


def _ins_body(g_ref, _, o_ref):
    o_ref[...] = g_ref[...]


def _tc_insert_tokens(gathered3d, partial_out):
    return pl.pallas_call(
        _ins_body,
        grid=(L,),
        in_specs=[
            pl.BlockSpec((1, B, TOKEN_DIM), lambda i: (i, 0, 0)),
            pl.BlockSpec(memory_space=pl.ANY),
        ],
        out_specs=pl.BlockSpec((1, B, TOKEN_DIM), lambda i: (i, 0, 0)),
        out_shape=jax.ShapeDtypeStruct((L, B, OUT_DIM), jnp.float32),
        input_output_aliases={1: 0},
    )(gathered3d, partial_out)


@jax.jit
def kernel(tokens, features, table, W, b):
    idx_flat = tokens.astype(jnp.int32).T.reshape(N_ROWS)
    gathered = _sc_gather(table, idx_flat)
    featT = jnp.transpose(features, (1, 0, 2))
    partial = _tc_project(featT, W, b.reshape(1, FEAT_EMB_DIM))
    outT = _tc_insert_tokens(gathered.reshape(L, B, TOKEN_DIM), partial)
    return jnp.transpose(outT, (1, 0, 2))


# speedup vs baseline: 2.4952x; 1.3090x over previous
import functools

import jax
import jax.numpy as jnp
from jax import lax
from jax.experimental import pallas as pl
from jax.experimental.pallas import tpu as pltpu
from jax.experimental.pallas import tpu_sc as plsc

VOCAB = 100000
TOKEN_DIM = 128
FEAT_DIM = 512
FEAT_EMB_DIM = 128
OUT_DIM = TOKEN_DIM + FEAT_EMB_DIM
B, L = 1024, 50
N_ROWS = B * L

NC = 2
NS = 16
NW = NC * NS
ROWS_PER_W = N_ROWS // NW
CHUNK = 128
NBUF = 4
SIZES = [CHUNK] * (ROWS_PER_W // CHUNK) + (
    [ROWS_PER_W % CHUNK] if ROWS_PER_W % CHUNK else []
)


def _sc_gather(table, idx_flat):
    mesh = plsc.VectorSubcoreMesh(core_axis_name="c", subcore_axis_name="s")

    @functools.partial(
        pl.kernel,
        mesh=mesh,
        out_type=jax.ShapeDtypeStruct((N_ROWS, TOKEN_DIM), jnp.float32),
        scratch_types=[
            pltpu.VMEM((ROWS_PER_W,), jnp.int32),
            pltpu.VMEM((NBUF, CHUNK, TOKEN_DIM), jnp.float32),
            [pltpu.SemaphoreType.DMA] * NBUF,
            [pltpu.SemaphoreType.DMA] * NBUF,
        ],
    )
    def gather_k(table_hbm, idx_hbm, out_hbm, idx_v, rows, gsems, wsems):
        wid = lax.axis_index("s") * NC + lax.axis_index("c")
        base = wid * ROWS_PER_W
        pltpu.sync_copy(idx_hbm.at[pl.ds(base, ROWS_PER_W)], idx_v)

        gathers = []
        writes = []
        for c, sz in enumerate(SIZES):
            off = c * CHUNK
            s = c % NBUF
            if c >= NBUF:
                writes[c - NBUF].wait()
            gcp = pltpu.make_async_copy(
                table_hbm.at[idx_v.at[pl.ds(off, sz)]],
                rows.at[s, pl.ds(0, sz)],
                gsems[s],
            )
            gcp.start()
            gathers.append(gcp)
            if c >= 1:
                ps = (c - 1) % NBUF
                psz = SIZES[c - 1]
                gathers[c - 1].wait()
                wcp = pltpu.make_async_copy(
                    rows.at[ps, pl.ds(0, psz)],
                    out_hbm.at[pl.ds(base + (c - 1) * CHUNK, psz)],
                    wsems[ps],
                )
                wcp.start()
                writes.append(wcp)
        n = len(SIZES)
        gathers[-1].wait()
        lcp = pltpu.make_async_copy(
            rows.at[(n - 1) % NBUF, pl.ds(0, SIZES[-1])],
            out_hbm.at[pl.ds(base + (n - 1) * CHUNK, SIZES[-1])],
            wsems[(n - 1) % NBUF],
        )
        lcp.start()
        writes.append(lcp)
        for c in range(max(0, n - NBUF), n):
            writes[c].wait()

    return gather_k(table, idx_flat)


LMM = 2
LIN = 5


def _mm_body(f_ref, w_ref, b_ref, o_ref):
    acc = lax.dot_general(
        f_ref[...].reshape(LMM * B, FEAT_DIM),
        w_ref[...],
        (((1,), (1,)), ((), ())),
        preferred_element_type=jnp.float32,
    )
    o_ref[...] = (acc + b_ref[...]).reshape(LMM, B, FEAT_EMB_DIM)


def _tc_project(featT, W, b2d):
    return pl.pallas_call(
        _mm_body,
        grid=(L // LMM,),
        in_specs=[
            pl.BlockSpec((LMM, B, FEAT_DIM), lambda i: (i, 0, 0)),
            pl.BlockSpec((FEAT_EMB_DIM, FEAT_DIM), lambda i: (0, 0)),
            pl.BlockSpec((1, FEAT_EMB_DIM), lambda i: (0, 0)),
        ],
        out_specs=pl.BlockSpec((LMM, B, FEAT_EMB_DIM), lambda i: (i, 0, 1)),
        out_shape=jax.ShapeDtypeStruct((L, B, OUT_DIM), jnp.float32),
    )(featT, W, b2d)


def _ins_body(g_ref, _, o_ref):
    o_ref[...] = g_ref[...]


def _tc_insert_tokens(gathered3d, partial_out):
    return pl.pallas_call(
        _ins_body,
        grid=(L // LIN,),
        in_specs=[
            pl.BlockSpec((LIN, B, TOKEN_DIM), lambda i: (i, 0, 0)),
            pl.BlockSpec(memory_space=pl.ANY),
        ],
        out_specs=pl.BlockSpec((LIN, B, TOKEN_DIM), lambda i: (i, 0, 0)),
        out_shape=jax.ShapeDtypeStruct((L, B, OUT_DIM), jnp.float32),
        input_output_aliases={1: 0},
    )(gathered3d, partial_out)


@jax.jit
def kernel(tokens, features, table, W, b):
    idx_flat = tokens.astype(jnp.int32).T.reshape(N_ROWS)
    gathered = _sc_gather(table, idx_flat)
    featT = jnp.transpose(features, (1, 0, 2))
    partial = _tc_project(featT, W, b.reshape(1, FEAT_EMB_DIM))
    outT = _tc_insert_tokens(gathered.reshape(L, B, TOKEN_DIM), partial)
    return jnp.transpose(outT, (1, 0, 2))


# speedup vs baseline: 2.6173x; 1.0490x over previous
import functools

import jax
import jax.numpy as jnp
from jax import lax
from jax.experimental import pallas as pl
from jax.experimental.pallas import tpu as pltpu
from jax.experimental.pallas import tpu_sc as plsc

VOCAB = 100000
TOKEN_DIM = 128
FEAT_DIM = 512
FEAT_EMB_DIM = 128
OUT_DIM = TOKEN_DIM + FEAT_EMB_DIM
B, L = 1024, 50
N_ROWS = B * L

NC = 2
NS = 16
NW = NC * NS
ROWS_PER_W = N_ROWS // NW
CHUNK = 128
NBUF = 4
SIZES = [CHUNK] * (ROWS_PER_W // CHUNK) + (
    [ROWS_PER_W % CHUNK] if ROWS_PER_W % CHUNK else []
)


def _sc_gather(table, idx_flat):
    mesh = plsc.VectorSubcoreMesh(core_axis_name="c", subcore_axis_name="s")

    @functools.partial(
        pl.kernel,
        mesh=mesh,
        out_type=jax.ShapeDtypeStruct((N_ROWS, TOKEN_DIM), jnp.float32),
        scratch_types=[
            pltpu.VMEM((ROWS_PER_W,), jnp.int32),
            pltpu.VMEM((NBUF, CHUNK, TOKEN_DIM), jnp.float32),
            [pltpu.SemaphoreType.DMA] * NBUF,
            [pltpu.SemaphoreType.DMA] * NBUF,
        ],
    )
    def gather_k(table_hbm, idx_hbm, out_hbm, idx_v, rows, gsems, wsems):
        wid = lax.axis_index("s") * NC + lax.axis_index("c")
        base = wid * ROWS_PER_W
        pltpu.sync_copy(idx_hbm.at[pl.ds(base, ROWS_PER_W)], idx_v)

        gathers = []
        writes = []
        for c, sz in enumerate(SIZES):
            off = c * CHUNK
            s = c % NBUF
            if c >= NBUF:
                writes[c - NBUF].wait()
            gcp = pltpu.make_async_copy(
                table_hbm.at[idx_v.at[pl.ds(off, sz)]],
                rows.at[s, pl.ds(0, sz)],
                gsems[s],
            )
            gcp.start()
            gathers.append(gcp)
            if c >= 1:
                ps = (c - 1) % NBUF
                psz = SIZES[c - 1]
                gathers[c - 1].wait()
                wcp = pltpu.make_async_copy(
                    rows.at[ps, pl.ds(0, psz)],
                    out_hbm.at[pl.ds(base + (c - 1) * CHUNK, psz)],
                    wsems[ps],
                )
                wcp.start()
                writes.append(wcp)
        n = len(SIZES)
        gathers[-1].wait()
        lcp = pltpu.make_async_copy(
            rows.at[(n - 1) % NBUF, pl.ds(0, SIZES[-1])],
            out_hbm.at[pl.ds(base + (n - 1) * CHUNK, SIZES[-1])],
            wsems[(n - 1) % NBUF],
        )
        lcp.start()
        writes.append(lcp)
        for c in range(max(0, n - NBUF), n):
            writes[c].wait()

    return gather_k(table, idx_flat)


LMM = 5
LIN = 10


def _mm_body(f_ref, w_ref, b_ref, o_ref):
    acc = lax.dot_general(
        f_ref[...].reshape(LMM * B, FEAT_DIM),
        w_ref[...],
        (((1,), (1,)), ((), ())),
        preferred_element_type=jnp.float32,
    )
    o_ref[...] = (acc + b_ref[...]).reshape(LMM, B, FEAT_EMB_DIM)


def _tc_project(featT, W, b2d):
    return pl.pallas_call(
        _mm_body,
        grid=(L // LMM,),
        in_specs=[
            pl.BlockSpec((LMM, B, FEAT_DIM), lambda i: (i, 0, 0)),
            pl.BlockSpec((FEAT_EMB_DIM, FEAT_DIM), lambda i: (0, 0)),
            pl.BlockSpec((1, FEAT_EMB_DIM), lambda i: (0, 0)),
        ],
        out_specs=pl.BlockSpec((LMM, B, FEAT_EMB_DIM), lambda i: (i, 0, 1)),
        out_shape=jax.ShapeDtypeStruct((L, B, OUT_DIM), jnp.float32),
    )(featT, W, b2d)


def _ins_body(g_ref, _, o_ref):
    o_ref[...] = g_ref[...]


def _tc_insert_tokens(gathered3d, partial_out):
    return pl.pallas_call(
        _ins_body,
        grid=(L // LIN,),
        in_specs=[
            pl.BlockSpec((LIN, B, TOKEN_DIM), lambda i: (i, 0, 0)),
            pl.BlockSpec(memory_space=pl.ANY),
        ],
        out_specs=pl.BlockSpec((LIN, B, TOKEN_DIM), lambda i: (i, 0, 0)),
        out_shape=jax.ShapeDtypeStruct((L, B, OUT_DIM), jnp.float32),
        input_output_aliases={1: 0},
    )(gathered3d, partial_out)


@jax.jit
def kernel(tokens, features, table, W, b):
    idx_flat = tokens.astype(jnp.int32).T.reshape(N_ROWS)
    gathered = _sc_gather(table, idx_flat)
    featT = jnp.transpose(features, (1, 0, 2))
    partial = _tc_project(featT, W, b.reshape(1, FEAT_EMB_DIM))
    outT = _tc_insert_tokens(gathered.reshape(L, B, TOKEN_DIM), partial)
    return jnp.transpose(outT, (1, 0, 2))


# speedup vs baseline: 2.6255x; 1.0031x over previous
import functools

import jax
import jax.numpy as jnp
from jax import lax
from jax.experimental import pallas as pl
from jax.experimental.pallas import tpu as pltpu
from jax.experimental.pallas import tpu_sc as plsc

VOCAB = 100000
TOKEN_DIM = 128
FEAT_DIM = 512
FEAT_EMB_DIM = 128
OUT_DIM = TOKEN_DIM + FEAT_EMB_DIM
B, L = 1024, 50
N_ROWS = B * L

NC = 2
NS = 16
NW = NC * NS
ROWS_PER_W = N_ROWS // NW
CHUNK = 128
NBUF = 4
SIZES = [CHUNK] * (ROWS_PER_W // CHUNK) + (
    [ROWS_PER_W % CHUNK] if ROWS_PER_W % CHUNK else []
)


def _sc_gather(table, idx_flat):
    mesh = plsc.VectorSubcoreMesh(core_axis_name="c", subcore_axis_name="s")

    @functools.partial(
        pl.kernel,
        mesh=mesh,
        out_type=jax.ShapeDtypeStruct((N_ROWS, TOKEN_DIM), jnp.float32),
        scratch_types=[
            pltpu.VMEM((ROWS_PER_W,), jnp.int32),
            pltpu.VMEM((NBUF, CHUNK, TOKEN_DIM), jnp.float32),
            [pltpu.SemaphoreType.DMA] * NBUF,
            [pltpu.SemaphoreType.DMA] * NBUF,
        ],
    )
    def gather_k(table_hbm, idx_hbm, out_hbm, idx_v, rows, gsems, wsems):
        wid = lax.axis_index("s") * NC + lax.axis_index("c")
        base = wid * ROWS_PER_W
        pltpu.sync_copy(idx_hbm.at[pl.ds(base, ROWS_PER_W)], idx_v)

        gathers = []
        writes = []
        for c, sz in enumerate(SIZES):
            off = c * CHUNK
            s = c % NBUF
            if c >= NBUF:
                writes[c - NBUF].wait()
            gcp = pltpu.make_async_copy(
                table_hbm.at[idx_v.at[pl.ds(off, sz)]],
                rows.at[s, pl.ds(0, sz)],
                gsems[s],
            )
            gcp.start()
            gathers.append(gcp)
            if c >= 1:
                ps = (c - 1) % NBUF
                psz = SIZES[c - 1]
                gathers[c - 1].wait()
                wcp = pltpu.make_async_copy(
                    rows.at[ps, pl.ds(0, psz)],
                    out_hbm.at[pl.ds(base + (c - 1) * CHUNK, psz)],
                    wsems[ps],
                )
                wcp.start()
                writes.append(wcp)
        n = len(SIZES)
        gathers[-1].wait()
        lcp = pltpu.make_async_copy(
            rows.at[(n - 1) % NBUF, pl.ds(0, SIZES[-1])],
            out_hbm.at[pl.ds(base + (n - 1) * CHUNK, SIZES[-1])],
            wsems[(n - 1) % NBUF],
        )
        lcp.start()
        writes.append(lcp)
        for c in range(max(0, n - NBUF), n):
            writes[c].wait()

    return gather_k(table, idx_flat)


LMM = 10
LIN = 25


def _mm_body(f_ref, w_ref, b_ref, o_ref):
    acc = lax.dot_general(
        f_ref[...].reshape(LMM * B, FEAT_DIM),
        w_ref[...],
        (((1,), (1,)), ((), ())),
        preferred_element_type=jnp.float32,
    )
    o_ref[...] = (acc + b_ref[...]).reshape(LMM, B, FEAT_EMB_DIM)


def _tc_project(featT, W, b2d):
    return pl.pallas_call(
        _mm_body,
        grid=(L // LMM,),
        in_specs=[
            pl.BlockSpec((LMM, B, FEAT_DIM), lambda i: (i, 0, 0)),
            pl.BlockSpec((FEAT_EMB_DIM, FEAT_DIM), lambda i: (0, 0)),
            pl.BlockSpec((1, FEAT_EMB_DIM), lambda i: (0, 0)),
        ],
        out_specs=pl.BlockSpec((LMM, B, FEAT_EMB_DIM), lambda i: (i, 0, 1)),
        out_shape=jax.ShapeDtypeStruct((L, B, OUT_DIM), jnp.float32),
    )(featT, W, b2d)


def _ins_body(g_ref, _, o_ref):
    o_ref[...] = g_ref[...]


def _tc_insert_tokens(gathered3d, partial_out):
    return pl.pallas_call(
        _ins_body,
        grid=(L // LIN,),
        in_specs=[
            pl.BlockSpec((LIN, B, TOKEN_DIM), lambda i: (i, 0, 0)),
            pl.BlockSpec(memory_space=pl.ANY),
        ],
        out_specs=pl.BlockSpec((LIN, B, TOKEN_DIM), lambda i: (i, 0, 0)),
        out_shape=jax.ShapeDtypeStruct((L, B, OUT_DIM), jnp.float32),
        input_output_aliases={1: 0},
    )(gathered3d, partial_out)


@jax.jit
def kernel(tokens, features, table, W, b):
    idx_flat = tokens.astype(jnp.int32).T.reshape(N_ROWS)
    gathered = _sc_gather(table, idx_flat)
    featT = jnp.transpose(features, (1, 0, 2))
    partial = _tc_project(featT, W, b.reshape(1, FEAT_EMB_DIM))
    outT = _tc_insert_tokens(gathered.reshape(L, B, TOKEN_DIM), partial)
    return jnp.transpose(outT, (1, 0, 2))
